# baseline (speedup 1.0000x reference)
import functools

import jax
import jax.numpy as jnp
from jax import lax
from jax.experimental import pallas as pl
from jax.experimental.pallas import tpu as pltpu
from jax.experimental.pallas import tpu_sc as plsc

_NUM_CORES = 2
_NUM_SUBCORES = 16
_NUM_WORKERS = _NUM_CORES * _NUM_SUBCORES
_CHUNK_ROWS = 16
_NBUF = 6
_WDEPTH = 2


def _make_copy(n_rows: int, emb: int):
  rows_per_w = n_rows // _NUM_WORKERS
  n_chunks = rows_per_w // _CHUNK_ROWS
  mesh = plsc.VectorSubcoreMesh(core_axis_name="c", subcore_axis_name="s")

  @functools.partial(
      pl.kernel,
      out_type=jax.ShapeDtypeStruct((n_rows, emb), jnp.float32),
      mesh=mesh,
      scratch_types=[
          pltpu.VMEM((_NBUF, _CHUNK_ROWS, emb), jnp.float32),
          [pltpu.SemaphoreType.DMA] * _NBUF,
          [pltpu.SemaphoreType.DMA] * _NBUF,
      ],
  )
  def copy_kernel(table_hbm, out_hbm, buf, rsems, wsems):
    wid = lax.axis_index("s") * _NUM_CORES + lax.axis_index("c")
    base = wid * rows_per_w

    def rd(c):
      return pltpu.make_async_copy(
          table_hbm.at[pl.ds(base + c * _CHUNK_ROWS, _CHUNK_ROWS)],
          buf.at[c % _NBUF],
          rsems[c % _NBUF],
      )

    def wr(c):
      return pltpu.make_async_copy(
          buf.at[c % _NBUF],
          out_hbm.at[pl.ds(base + c * _CHUNK_ROWS, _CHUNK_ROWS)],
          wsems[c % _NBUF],
      )

    prime = min(_NBUF - _WDEPTH, n_chunks)
    for c in range(prime):
      rd(c).start()
    for c in range(n_chunks):
      rd(c).wait()
      wr(c).start()
      if c >= _WDEPTH:
        wr(c - _WDEPTH).wait()
      if c + prime < n_chunks:
        rd(c + prime).start()
    for c in range(max(0, n_chunks - _WDEPTH), n_chunks):
      wr(c).wait()

  return copy_kernel


def kernel(x, table):
  n_rows, emb = table.shape
  seq_len = x.shape[1]
  out = _make_copy(seq_len, emb)(table[:seq_len])
  return out[None]


# speedup vs baseline: 1.0107x; 1.0107x over previous
import functools

import jax
import jax.numpy as jnp
from jax import lax
from jax.experimental import pallas as pl
from jax.experimental.pallas import tpu as pltpu
from jax.experimental.pallas import tpu_sc as plsc

_NUM_CORES = 2
_NUM_SUBCORES = 16
_NUM_WORKERS = _NUM_CORES * _NUM_SUBCORES
_CHUNK_ROWS = 16
_NBUF = 3


def _make_copy(n_rows: int, emb: int):
  rows_per_w = n_rows // _NUM_WORKERS
  n_chunks = rows_per_w // _CHUNK_ROWS
  n_j = n_chunks // 2
  mesh = plsc.VectorSubcoreMesh(core_axis_name="c", subcore_axis_name="s")

  @functools.partial(
      pl.kernel,
      out_type=jax.ShapeDtypeStruct((n_rows, emb), jnp.float32),
      mesh=mesh,
      scratch_types=[
          pltpu.VMEM((_NBUF, _CHUNK_ROWS, emb), jnp.float32),
          pltpu.VMEM_SHARED((_NUM_SUBCORES, _NBUF, _CHUNK_ROWS, emb),
                            jnp.float32),
          [pltpu.SemaphoreType.DMA] * _NBUF,
          [pltpu.SemaphoreType.DMA] * _NBUF,
          [pltpu.SemaphoreType.DMA] * _NBUF,
          [pltpu.SemaphoreType.DMA] * _NBUF,
      ],
  )
  def copy_kernel(table_hbm, out_hbm, buf_a, buf_b, rs_a, ws_a, rs_b, ws_b):
    cid = lax.axis_index("c")
    sid = lax.axis_index("s")
    wid = sid * _NUM_CORES + cid
    base = wid * rows_per_w

    def mk(path):
      def chunk_slice(j):
        c = 2 * j + path
        return pl.ds(base + c * _CHUNK_ROWS, _CHUNK_ROWS)

      def bget(j):
        b = j % _NBUF
        return buf_a.at[b] if path == 0 else buf_b.at[sid, b]

      rs = rs_a if path == 0 else rs_b
      ws = ws_a if path == 0 else ws_b

      def rd(j):
        return pltpu.make_async_copy(
            table_hbm.at[chunk_slice(j)], bget(j), rs[j % _NBUF])

      def wr(j):
        return pltpu.make_async_copy(
            bget(j), out_hbm.at[chunk_slice(j)], ws[j % _NBUF])

      return rd, wr

    rd_a, wr_a = mk(0)
    rd_b, wr_b = mk(1)
    prime = min(_NBUF - 1, n_j)
    for j in range(prime):
      rd_a(j).start()
      rd_b(j).start()
    for j in range(n_j):
      for rd, wr in ((rd_a, wr_a), (rd_b, wr_b)):
        rd(j).wait()
        wr(j).start()
        if j >= 1:
          wr(j - 1).wait()
        if j + prime < n_j:
          rd(j + prime).start()
    wr_a(n_j - 1).wait()
    wr_b(n_j - 1).wait()

  return copy_kernel


def kernel(x, table):
  n_rows, emb = table.shape
  seq_len = x.shape[1]
  out = _make_copy(seq_len, emb)(table[:seq_len])
  return out[None]
